# baseline (speedup 1.0000x reference)
import jax
import jax.numpy as jnp
from jax import lax
from jax.experimental import pallas as pl
from jax.experimental.pallas import tpu as pltpu
from jax.experimental.pallas import tpu_sc as plsc

_B = 16384
_D = 128
_L = 16
_NW = 32
_BPW = _B // _NW
_C = 128
_NCHUNK = _BPW // _C


def _dist_mult_body(h_hbm, t_hbm, r_hbm, tab_hbm, out_hbm,
                    idx_v, h_v, t_v, rel_v, out_v, gsem):
    cid = lax.axis_index("c")
    sid = lax.axis_index("s")
    wid = sid * 2 + cid
    base = wid * _BPW
    lane = lax.iota(jnp.int32, _L)

    def chunk_body(c, carry):
        off = base + c * _C
        pltpu.sync_copy(r_hbm.at[pl.ds(off, _C)], idx_v)
        gather = pltpu.async_copy(tab_hbm.at[idx_v], rel_v, gsem)
        pltpu.sync_copy(h_hbm.at[pl.ds(off, _C), :], h_v)
        pltpu.sync_copy(t_hbm.at[pl.ds(off, _C), :], t_v)
        gather.wait()

        def group_body(g, carry2):
            row = g * _L + lane
            accs = [jnp.zeros((_L,), jnp.float32) for _ in range(4)]
            for d in range(_D):
                col = jnp.full((_L,), d, jnp.int32)
                term = (plsc.load_gather(h_v, [row, col])
                        * plsc.load_gather(rel_v, [row, col])
                        * plsc.load_gather(t_v, [row, col]))
                accs[d % 4] = accs[d % 4] + term
            out_v[pl.ds(g * _L, _L)] = (accs[0] + accs[1]) + (accs[2] + accs[3])
            return carry2

        lax.fori_loop(0, _C // _L, group_body, 0)
        pltpu.sync_copy(out_v, out_hbm.at[pl.ds(off, _C)])
        return carry

    lax.fori_loop(0, _NCHUNK, chunk_body, 0)


@jax.jit
def _dist_mult(h, t, r, rel_emb):
    mesh = plsc.VectorSubcoreMesh(core_axis_name="c", subcore_axis_name="s")
    run = pl.kernel(
        _dist_mult_body,
        out_type=jax.ShapeDtypeStruct((_B,), jnp.float32),
        mesh=mesh,
        compiler_params=pltpu.CompilerParams(needs_layout_passes=False),
        scratch_types=[
            pltpu.VMEM((_C,), jnp.int32),
            pltpu.VMEM((_C, _D), jnp.float32),
            pltpu.VMEM((_C, _D), jnp.float32),
            pltpu.VMEM((_C, _D), jnp.float32),
            pltpu.VMEM((_C,), jnp.float32),
            pltpu.SemaphoreType.DMA,
        ],
    )
    return run(h, t, r, rel_emb)


def kernel(h, t, r, rel_emb):
    return _dist_mult(h, t, r.astype(jnp.int32), rel_emb)


# speedup vs baseline: 3.9321x; 3.9321x over previous
import jax
import jax.numpy as jnp
from jax import lax
from jax.experimental import pallas as pl
from jax.experimental.pallas import tpu as pltpu
from jax.experimental.pallas import tpu_sc as plsc

_B = 16384
_D = 128
_L = 16
_NW = 32
_BPW = _B // _NW
_C = 128
_NCHUNK = _BPW // _C


def _dist_mult_body(h_hbm, t_hbm, r_hbm, tab_hbm, out_hbm,
                    idx_v, h_v, t_v, rel_v, out_v,
                    isem, sem0, sem1, osem):
    cid = lax.axis_index("c")
    sid = lax.axis_index("s")
    wid = sid * 2 + cid
    base = wid * _BPW
    lane = lax.iota(jnp.int32, _L)
    last = lane == (_L - 1)
    sems = (sem0, sem1)

    idx_copies = [
        pltpu.async_copy(r_hbm.at[pl.ds(base + c * _C, _C)], idx_v.at[c], isem)
        for c in range(_NCHUNK)
    ]
    for cp in idx_copies:
        cp.wait()

    def fetch(c, b):
        off = base + c * _C
        return [
            pltpu.async_copy(tab_hbm.at[idx_v.at[c]], rel_v.at[b], sems[b]),
            pltpu.async_copy(h_hbm.at[pl.ds(off, _C), :], h_v.at[b], sems[b]),
            pltpu.async_copy(t_hbm.at[pl.ds(off, _C), :], t_v.at[b], sems[b]),
        ]

    pending = fetch(0, 0)
    out_copies = [None, None]
    for c in range(_NCHUNK):
        b = c % 2
        nxt = fetch(c + 1, 1 - b) if c + 1 < _NCHUNK else None
        for cp in pending:
            cp.wait()
        pending = nxt
        if out_copies[b] is not None:
            out_copies[b].wait()

        def row_body(i, carry, _b=b):
            acc = jnp.zeros((_L,), jnp.float32)
            for j in range(_D // _L):
                sl = pl.ds(j * _L, _L)
                acc = acc + (h_v[_b, i, sl] * rel_v[_b, i, sl]) * t_v[_b, i, sl]
            cum = plsc.cumsum(acc)
            plsc.store_compressed(out_v.at[_b, pl.ds(i, _L)], cum, mask=last)
            return carry

        lax.fori_loop(0, _C, row_body, 0, unroll=4)
        out_copies[b] = pltpu.async_copy(
            out_v.at[b, pl.ds(0, _C)], out_hbm.at[pl.ds(base + c * _C, _C)],
            osem)
    for cp in out_copies:
        if cp is not None:
            cp.wait()


@jax.jit
def _dist_mult(h, t, r, rel_emb):
    mesh = plsc.VectorSubcoreMesh(core_axis_name="c", subcore_axis_name="s")
    run = pl.kernel(
        _dist_mult_body,
        out_type=jax.ShapeDtypeStruct((_B,), jnp.float32),
        mesh=mesh,
        compiler_params=pltpu.CompilerParams(needs_layout_passes=False),
        scratch_types=[
            pltpu.VMEM((_NCHUNK, _C), jnp.int32),
            pltpu.VMEM((2, _C, _D), jnp.float32),
            pltpu.VMEM((2, _C, _D), jnp.float32),
            pltpu.VMEM((2, _C, _D), jnp.float32),
            pltpu.VMEM((2, _C + _L), jnp.float32),
            pltpu.SemaphoreType.DMA,
            pltpu.SemaphoreType.DMA,
            pltpu.SemaphoreType.DMA,
            pltpu.SemaphoreType.DMA,
        ],
    )
    return run(h, t, r, rel_emb)


def kernel(h, t, r, rel_emb):
    return _dist_mult(h, t, r.astype(jnp.int32), rel_emb)
